# Initial kernel scaffold; baseline (speedup 1.0000x reference)
import jax
import jax.numpy as jnp
from jax.experimental import pallas as pl


def kernel(query_id, search_source, click_item_id, click_item_category, query_words, item_id_emb, item_cat_emb, query_id_emb, qsrc_emb, qword_emb, W_q, b_q, W_i, b_i):
    raise NotImplementedError("write your pallas kernel here")


# speedup vs baseline: 2.9414x; 2.9414x over previous
import functools

import jax
import jax.numpy as jnp
from jax import lax
from jax.experimental import pallas as pl
from jax.experimental.pallas import tpu as pltpu
from jax.experimental.pallas import tpu_sc as plsc

B, L, NW, NC = 1024, 50, 8, 3
T = B * L
CHUNK = 128
NCHUNK = T // CHUNK
NWORK = 32
KMAX = -(-NCHUNK // NWORK)
D_ID, D_CAT, D_SRC, D_W = 64, 32, 16, 64
ITEM_SIZE = 96


def _sc_body(qid_i, qsrc_i, qw_i, ii_i, ic_i,
             qid_t, qw_t, qsrc_t, iid_t, icat_t, zeros_in,
             qid_o, qwsum_o, qsrc_o, iid_o, icat_o,
             qid_idx, qsrc_idx, qw_idx, ii_idx, ic_idx, dst_idx,
             qid_rows, qsrc_rows, qw_rows, ii_rows, ic_rows,
             pooled_sh, sem_idx, sem_qw, sem_g, sem_sa, sem_out):
    cid = lax.axis_index("c")
    sid = lax.axis_index("s")
    wid = sid * 2 + cid

    lanes = lax.broadcasted_iota(jnp.int32, (16,), 0)

    for j in range(8):
        for v in range(8):
            vals = sid * 128 + 16 * j + 2 * v + (lanes >> 3)
            dst_idx[j, pl.ds(16 * v, 16)] = vals

    @pl.loop(0, KMAX)
    def _chunk_loop(k):
        chunk = k * NWORK + wid

        @pl.when(chunk < NCHUNK)
        def _():
            tok0 = chunk * CHUNK
            c1 = pltpu.async_copy(qid_i.at[pl.ds(tok0, 128)], qid_idx, sem_idx)
            c2 = pltpu.async_copy(qsrc_i.at[pl.ds(tok0, 128)], qsrc_idx,
                                  sem_idx)
            c3 = pltpu.async_copy(qw_i.at[pl.ds(tok0 * 8, 1024)], qw_idx,
                                  sem_idx)
            c4 = pltpu.async_copy(ii_i.at[pl.ds(tok0 * 3, 384)], ii_idx,
                                  sem_idx)
            c5 = pltpu.async_copy(ic_i.at[pl.ds(tok0 * 3, 384)], ic_idx,
                                  sem_idx)
            z = pltpu.async_copy(zeros_in,
                                 pooled_sh.at[pl.ds(sid * 128, 128)], sem_sa)
            c1.wait(); c2.wait(); c3.wait(); c4.wait(); c5.wait()
            gq = [pltpu.async_copy(qw_t.at[qw_idx.at[pl.ds(j * 128, 128)]],
                                   qw_rows.at[pl.ds(j * 128, 128)], sem_qw)
                  for j in range(8)]
            g1 = pltpu.async_copy(qid_t.at[qid_idx], qid_rows, sem_g)
            g2 = pltpu.async_copy(qsrc_t.at[qsrc_idx], qsrc_rows, sem_g)
            gi = [pltpu.async_copy(iid_t.at[ii_idx.at[pl.ds(j * 128, 128)]],
                                   ii_rows.at[pl.ds(j * 128, 128)], sem_g)
                  for j in range(3)]
            gc = [pltpu.async_copy(icat_t.at[ic_idx.at[pl.ds(j * 128, 128)]],
                                   ic_rows.at[pl.ds(j * 128, 128)], sem_g)
                  for j in range(3)]
            z.wait()
            for g in gq:
                g.wait()
            sa = [pltpu.async_copy(qw_rows.at[pl.ds(j * 128, 128)],
                                   pooled_sh.at[dst_idx.at[j]], sem_sa,
                                   add=True)
                  for j in range(8)]
            g1.wait(); g2.wait()
            for g in gi:
                g.wait()
            for g in gc:
                g.wait()
            o1 = pltpu.async_copy(qid_rows, qid_o.at[pl.ds(tok0, CHUNK)],
                                  sem_out)
            o2 = pltpu.async_copy(qsrc_rows, qsrc_o.at[pl.ds(tok0, CHUNK)],
                                  sem_out)
            o3 = pltpu.async_copy(ii_rows, iid_o.at[pl.ds(chunk * 384, 384)],
                                  sem_out)
            o4 = pltpu.async_copy(ic_rows, icat_o.at[pl.ds(chunk * 384, 384)],
                                  sem_out)
            for s in sa:
                s.wait()
            o5 = pltpu.async_copy(pooled_sh.at[pl.ds(sid * 128, 128)],
                                  qwsum_o.at[pl.ds(tok0, CHUNK)], sem_out)
            o1.wait(); o2.wait(); o3.wait(); o4.wait(); o5.wait()


def _sc_gather(qid2d, qsrc2d, qw2d, ii2d, ic2d,
               qid_emb, qword_emb, qsrc_emb, iid_emb, icat_emb):
    mesh = plsc.VectorSubcoreMesh(core_axis_name="c", subcore_axis_name="s",
                                  num_cores=2, num_subcores=16)
    f32 = jnp.float32
    out_type = (
        jax.ShapeDtypeStruct((T, D_ID), f32),
        jax.ShapeDtypeStruct((T, D_W), f32),
        jax.ShapeDtypeStruct((T, D_SRC), f32),
        jax.ShapeDtypeStruct((T * NC, D_ID), f32),
        jax.ShapeDtypeStruct((T * NC, D_CAT), f32),
    )
    scratch = [
        pltpu.VMEM((128,), jnp.int32),
        pltpu.VMEM((128,), jnp.int32),
        pltpu.VMEM((1024,), jnp.int32),
        pltpu.VMEM((384,), jnp.int32),
        pltpu.VMEM((384,), jnp.int32),
        pltpu.VMEM((8, 128), jnp.int32),
        pltpu.VMEM((128, D_ID), f32),
        pltpu.VMEM((128, D_SRC), f32),
        pltpu.VMEM((1024, D_W), f32),
        pltpu.VMEM((384, D_ID), f32),
        pltpu.VMEM((384, D_CAT), f32),
        pltpu.VMEM_SHARED((16 * 128, D_W), f32),
        pltpu.SemaphoreType.DMA,
        pltpu.SemaphoreType.DMA,
        pltpu.SemaphoreType.DMA,
        pltpu.SemaphoreType.DMA,
        pltpu.SemaphoreType.DMA,
    ]
    fn = pl.kernel(_sc_body, out_type=out_type, mesh=mesh,
                   scratch_types=scratch,
                   compiler_params=pltpu.CompilerParams(
                       use_tc_tiling_on_sc=False))
    zeros_in = jnp.zeros((CHUNK, D_W), f32)
    return fn(qid2d, qsrc2d, qw2d, ii2d, ic2d,
              qid_emb, qword_emb, qsrc_emb, iid_emb, icat_emb, zeros_in)


QBLK = 512
IBLK = QBLK * NC
MROWS = IBLK // 128
GRID = T // QBLK


def _tc_body(qid_r, qsum_r, words_r, emb0_r, wqt_r, wqb_r, bq_r,
             iid_r, icat_r, wit_r, wib_r, bi_r, iidm_r,
             qout_r, iout_r, mask_r):
    words = words_r[...]
    wc = jnp.sum((words != 0).astype(jnp.float32), axis=1, keepdims=True)
    denom = jnp.maximum(wc, 1.0)
    pooled = (qsum_r[...] - (NW - wc) * emb0_r[...]) / denom
    qout_r[...] = (
        jnp.dot(qid_r[...], wqt_r[...], preferred_element_type=jnp.float32)
        + jnp.dot(pooled, wqb_r[...], preferred_element_type=jnp.float32)
        + bq_r[...]
    )
    iout_r[...] = (
        jnp.dot(iid_r[...], wit_r[...], preferred_element_type=jnp.float32)
        + jnp.dot(icat_r[...], wib_r[...], preferred_element_type=jnp.float32)
        + bi_r[...]
    )
    mask_r[...] = iidm_r[...] != 0


def _tc_project(qid_rows, qwsum, words2, emb0, wqt, wqb, bq,
                iid_rows, icat_rows, wit, wib, bi, iidm2d):
    f32 = jnp.float32
    full = lambda shape: pl.BlockSpec(shape, lambda g: (0, 0))
    return pl.pallas_call(
        _tc_body,
        grid=(GRID,),
        in_specs=[
            pl.BlockSpec((QBLK, D_ID), lambda g: (g, 0)),
            pl.BlockSpec((QBLK, D_W), lambda g: (g, 0)),
            pl.BlockSpec((QBLK, NW), lambda g: (g, 0)),
            full((1, D_W)),
            full((D_ID, ITEM_SIZE)),
            full((D_W, ITEM_SIZE)),
            full((1, ITEM_SIZE)),
            pl.BlockSpec((IBLK, D_ID), lambda g: (g, 0)),
            pl.BlockSpec((IBLK, D_CAT), lambda g: (g, 0)),
            full((D_ID, ITEM_SIZE)),
            full((D_CAT, ITEM_SIZE)),
            full((1, ITEM_SIZE)),
            pl.BlockSpec((1, 1, IBLK), lambda g: (g, 0, 0)),
        ],
        out_specs=[
            pl.BlockSpec((QBLK, ITEM_SIZE), lambda g: (g, 0)),
            pl.BlockSpec((IBLK, ITEM_SIZE), lambda g: (g, 0)),
            pl.BlockSpec((1, 1, IBLK), lambda g: (g, 0, 0)),
        ],
        out_shape=[
            jax.ShapeDtypeStruct((T, ITEM_SIZE), f32),
            jax.ShapeDtypeStruct((T * NC, ITEM_SIZE), f32),
            jax.ShapeDtypeStruct((GRID, 1, IBLK), jnp.bool_),
        ],
    )(qid_rows, qwsum, words2, emb0, wqt, wqb, bq,
      iid_rows, icat_rows, wit, wib, bi, iidm2d)


def kernel(query_id, search_source, click_item_id, click_item_category,
           query_words, item_id_emb, item_cat_emb, query_id_emb, qsrc_emb,
           qword_emb, W_q, b_q, W_i, b_i):
    i32 = jnp.int32
    qid2d = query_id.astype(i32).reshape(T)
    qsrc2d = search_source.astype(i32).reshape(T)
    qw2d = query_words.astype(i32).reshape(T * NW)
    ii2d = click_item_id.astype(i32).reshape(T * NC)
    ic2d = click_item_category.astype(i32).reshape(T * NC)

    qid_rows, qwsum, qsrc_rows, iid_rows, icat_rows = _sc_gather(
        qid2d, qsrc2d, qw2d, ii2d, ic2d,
        query_id_emb, qword_emb, qsrc_emb, item_id_emb, item_cat_emb)

    words2 = query_words.astype(i32).reshape(T, NW)
    iidm2d = click_item_id.astype(i32).reshape(GRID, 1, IBLK)
    emb0 = qword_emb[0:1, :]
    query_emb, item_out, mask2d = _tc_project(
        qid_rows, qwsum, words2, emb0, W_q[:D_ID], W_q[D_ID:],
        b_q.reshape(1, ITEM_SIZE), iid_rows, icat_rows, W_i[:D_ID],
        W_i[D_ID:], b_i.reshape(1, ITEM_SIZE), iidm2d)

    return (query_emb.reshape(B, L, ITEM_SIZE),
            qsrc_rows.reshape(B, L, D_SRC),
            item_out.reshape(B, L, NC, ITEM_SIZE),
            mask2d.reshape(B, L, NC))


# speedup vs baseline: 3.4742x; 1.1811x over previous
import functools

import jax
import jax.numpy as jnp
from jax import lax
from jax.experimental import pallas as pl
from jax.experimental.pallas import tpu as pltpu
from jax.experimental.pallas import tpu_sc as plsc

B, L, NW, NC = 1024, 50, 8, 3
T = B * L
CHUNK = 128
NCHUNK = T // CHUNK
NWORK = 32
KMAX = -(-NCHUNK // NWORK)
D_ID, D_CAT, D_SRC, D_W = 64, 32, 16, 64
ITEM_SIZE = 96


def _sc_body(qid_i, qsrc_i, qw_i, ii_i, ic_i,
             qid_t, qw_t, qsrc_t, iid_t, icat_t, zeros_in,
             qcat_o, qsrc_o, item_o,
             qid_idx, qsrc_idx, qw_idx, ii_idx, ic_idx, dst_idx,
             qid_rows, qsrc_rows, qw_rows, ii_rows, ic_rows,
             pooled_sh, sem_idx, sem_qw, sem_g, sem_sa, sem_out):
    cid = lax.axis_index("c")
    sid = lax.axis_index("s")
    wid = sid * 2 + cid

    lanes = lax.broadcasted_iota(jnp.int32, (16,), 0)

    for j in range(8):
        for v in range(8):
            vals = sid * 128 + 16 * j + 2 * v + (lanes >> 3)
            dst_idx[j, pl.ds(16 * v, 16)] = vals

    @pl.loop(0, KMAX)
    def _chunk_loop(k):
        chunk = k * NWORK + wid

        @pl.when(chunk < NCHUNK)
        def _():
            tok0 = chunk * CHUNK
            c1 = pltpu.async_copy(qid_i.at[pl.ds(tok0, 128)], qid_idx, sem_idx)
            c2 = pltpu.async_copy(qsrc_i.at[pl.ds(tok0, 128)], qsrc_idx,
                                  sem_idx)
            c3 = pltpu.async_copy(qw_i.at[pl.ds(tok0 * 8, 1024)], qw_idx,
                                  sem_idx)
            c4 = pltpu.async_copy(ii_i.at[pl.ds(tok0 * 3, 384)], ii_idx,
                                  sem_idx)
            c5 = pltpu.async_copy(ic_i.at[pl.ds(tok0 * 3, 384)], ic_idx,
                                  sem_idx)
            z = pltpu.async_copy(zeros_in,
                                 pooled_sh.at[pl.ds(sid * 128, 128)], sem_sa)
            c1.wait(); c2.wait(); c3.wait(); c4.wait(); c5.wait()
            gq = [pltpu.async_copy(qw_t.at[qw_idx.at[pl.ds(j * 128, 128)]],
                                   qw_rows.at[pl.ds(j * 128, 128)], sem_qw)
                  for j in range(8)]
            g1 = pltpu.async_copy(qid_t.at[qid_idx], qid_rows, sem_g)
            g2 = pltpu.async_copy(qsrc_t.at[qsrc_idx], qsrc_rows, sem_g)
            gi = [pltpu.async_copy(iid_t.at[ii_idx.at[pl.ds(j * 128, 128)]],
                                   ii_rows.at[pl.ds(j * 128, 128)], sem_g)
                  for j in range(3)]
            gc = [pltpu.async_copy(icat_t.at[ic_idx.at[pl.ds(j * 128, 128)]],
                                   ic_rows.at[pl.ds(j * 128, 128)], sem_g)
                  for j in range(3)]
            z.wait()
            for g in gq:
                g.wait()
            sa = [pltpu.async_copy(qw_rows.at[pl.ds(j * 128, 128)],
                                   pooled_sh.at[dst_idx.at[j]], sem_sa,
                                   add=True)
                  for j in range(8)]
            g1.wait(); g2.wait()
            for g in gi:
                g.wait()
            for g in gc:
                g.wait()
            o1 = pltpu.async_copy(
                qid_rows, qcat_o.at[pl.ds(tok0, CHUNK), pl.ds(0, D_ID)],
                sem_out)
            o2 = pltpu.async_copy(qsrc_rows, qsrc_o.at[pl.ds(tok0, CHUNK)],
                                  sem_out)
            o3 = pltpu.async_copy(
                ii_rows, item_o.at[pl.ds(chunk * 384, 384), pl.ds(0, D_ID)],
                sem_out)
            o4 = pltpu.async_copy(
                ic_rows,
                item_o.at[pl.ds(chunk * 384, 384), pl.ds(D_ID, D_CAT)],
                sem_out)
            for s in sa:
                s.wait()
            o5 = pltpu.async_copy(
                pooled_sh.at[pl.ds(sid * 128, 128)],
                qcat_o.at[pl.ds(tok0, CHUNK), pl.ds(D_ID, D_W)], sem_out)
            o1.wait(); o2.wait(); o3.wait(); o4.wait(); o5.wait()


def _sc_gather(qid2d, qsrc2d, qw2d, ii2d, ic2d,
               qid_emb, qword_emb, qsrc_emb, iid_emb, icat_emb):
    mesh = plsc.VectorSubcoreMesh(core_axis_name="c", subcore_axis_name="s",
                                  num_cores=2, num_subcores=16)
    f32 = jnp.float32
    out_type = (
        jax.ShapeDtypeStruct((T, 128), f32),
        jax.ShapeDtypeStruct((T, D_SRC), f32),
        jax.ShapeDtypeStruct((T * NC, 128), f32),
    )
    scratch = [
        pltpu.VMEM((128,), jnp.int32),
        pltpu.VMEM((128,), jnp.int32),
        pltpu.VMEM((1024,), jnp.int32),
        pltpu.VMEM((384,), jnp.int32),
        pltpu.VMEM((384,), jnp.int32),
        pltpu.VMEM((8, 128), jnp.int32),
        pltpu.VMEM((128, D_ID), f32),
        pltpu.VMEM((128, D_SRC), f32),
        pltpu.VMEM((1024, D_W), f32),
        pltpu.VMEM((384, D_ID), f32),
        pltpu.VMEM((384, D_CAT), f32),
        pltpu.VMEM_SHARED((16 * 128, D_W), f32),
        pltpu.SemaphoreType.DMA,
        pltpu.SemaphoreType.DMA,
        pltpu.SemaphoreType.DMA,
        pltpu.SemaphoreType.DMA,
        pltpu.SemaphoreType.DMA,
    ]
    fn = pl.kernel(_sc_body, out_type=out_type, mesh=mesh,
                   scratch_types=scratch,
                   compiler_params=pltpu.CompilerParams(
                       use_tc_tiling_on_sc=False))
    zeros_in = jnp.zeros((CHUNK, D_W), f32)
    return fn(qid2d, qsrc2d, qw2d, ii2d, ic2d,
              qid_emb, qword_emb, qsrc_emb, iid_emb, icat_emb, zeros_in)


QBLK = 512
IBLK = QBLK * NC
GRID = T // QBLK


def _tc_body(qcat_r, words_r, emb0p_r, cmask_r, wq_r, bq_r,
             item_r, wi_r, bi_r, iidm_r,
             qout_r, iout_r, mask_r):
    words = words_r[...]
    wc = jnp.sum((words != 0).astype(jnp.float32), axis=1, keepdims=True)
    rdenom = 1.0 / jnp.maximum(wc, 1.0)
    scale = 1.0 + (rdenom - 1.0) * cmask_r[...]
    corrected = qcat_r[...] * scale - ((NW - wc) * rdenom) * emb0p_r[...]
    qout_r[...] = (
        jnp.dot(corrected, wq_r[...], preferred_element_type=jnp.float32)
        + bq_r[...]
    )
    iout_r[...] = (
        jnp.dot(item_r[..., :ITEM_SIZE], wi_r[...],
                preferred_element_type=jnp.float32)
        + bi_r[...]
    )
    mask_r[...] = iidm_r[...] != 0


def _tc_project(qcat, words2, emb0p, cmask, wq, bq, item, wi, bi, iidm3d):
    f32 = jnp.float32
    full = lambda shape: pl.BlockSpec(shape, lambda g: (0, 0))
    return pl.pallas_call(
        _tc_body,
        grid=(GRID,),
        in_specs=[
            pl.BlockSpec((QBLK, 128), lambda g: (g, 0)),
            pl.BlockSpec((QBLK, NW), lambda g: (g, 0)),
            full((1, 128)),
            full((1, 128)),
            full((128, ITEM_SIZE)),
            full((1, ITEM_SIZE)),
            pl.BlockSpec((IBLK, 128), lambda g: (g, 0)),
            full((ITEM_SIZE, ITEM_SIZE)),
            full((1, ITEM_SIZE)),
            pl.BlockSpec((1, 1, IBLK), lambda g: (g, 0, 0)),
        ],
        out_specs=[
            pl.BlockSpec((QBLK, ITEM_SIZE), lambda g: (g, 0)),
            pl.BlockSpec((IBLK, ITEM_SIZE), lambda g: (g, 0)),
            pl.BlockSpec((1, 1, IBLK), lambda g: (g, 0, 0)),
        ],
        out_shape=[
            jax.ShapeDtypeStruct((T, ITEM_SIZE), f32),
            jax.ShapeDtypeStruct((T * NC, ITEM_SIZE), f32),
            jax.ShapeDtypeStruct((GRID, 1, IBLK), jnp.bool_),
        ],
    )(qcat, words2, emb0p, cmask, wq, bq, item, wi, bi, iidm3d)


def kernel(query_id, search_source, click_item_id, click_item_category,
           query_words, item_id_emb, item_cat_emb, query_id_emb, qsrc_emb,
           qword_emb, W_q, b_q, W_i, b_i):
    i32 = jnp.int32
    f32 = jnp.float32
    qid2d = query_id.astype(i32).reshape(T)
    qsrc2d = search_source.astype(i32).reshape(T)
    qw2d = query_words.astype(i32).reshape(T * NW)
    ii2d = click_item_id.astype(i32).reshape(T * NC)
    ic2d = click_item_category.astype(i32).reshape(T * NC)

    qcat, qsrc_rows, item_rows = _sc_gather(
        qid2d, qsrc2d, qw2d, ii2d, ic2d,
        query_id_emb, qword_emb, qsrc_emb, item_id_emb, item_cat_emb)

    words2 = query_words.astype(i32).reshape(T, NW)
    iidm3d = click_item_id.astype(i32).reshape(GRID, 1, IBLK)
    emb0p = jnp.concatenate(
        [jnp.zeros((1, D_ID), f32), qword_emb[0:1, :]], axis=1)
    cmask = jnp.concatenate(
        [jnp.zeros((1, D_ID), f32), jnp.ones((1, D_W), f32)], axis=1)
    query_emb, item_out, mask3d = _tc_project(
        qcat, words2, emb0p, cmask, W_q, b_q.reshape(1, ITEM_SIZE),
        item_rows, W_i, b_i.reshape(1, ITEM_SIZE), iidm3d)

    return (query_emb.reshape(B, L, ITEM_SIZE),
            qsrc_rows.reshape(B, L, D_SRC),
            item_out.reshape(B, L, NC, ITEM_SIZE),
            mask3d.reshape(B, L, NC))


# speedup vs baseline: 3.5864x; 1.0323x over previous
import functools

import jax
import jax.numpy as jnp
from jax import lax
from jax.experimental import pallas as pl
from jax.experimental.pallas import tpu as pltpu
from jax.experimental.pallas import tpu_sc as plsc

B, L, NW, NC = 1024, 50, 8, 3
T = B * L
CHUNK = 128
NCHUNK = T // CHUNK
NWORK = 32
KMAX = -(-NCHUNK // NWORK)
D_ID, D_CAT, D_SRC, D_W = 64, 32, 16, 64
ITEM_SIZE = 96


def _sc_body(qid_i, qsrc_i, qw_i, ii_i, ic_i,
             qid_t, qw_t, qsrc_t, iid_t, icat_t, zeros_in,
             qcat_o, qsrc_o, item_o,
             qid_idx, qsrc_idx, qw_idx, ii_idx, ic_idx, dst_idx,
             qid_rows, qsrc_rows, qw_rows, ii_rows, ic_rows,
             pooled_sh, sem_idx, sem_qw, sem_g, sem_sa, sem_out):
    cid = lax.axis_index("c")
    sid = lax.axis_index("s")
    wid = sid * 2 + cid

    lanes = lax.broadcasted_iota(jnp.int32, (16,), 0)

    for j in range(8):
        for v in range(8):
            vals = sid * 128 + 16 * j + 2 * v + (lanes >> 3)
            dst_idx[j, pl.ds(16 * v, 16)] = vals

    @pl.loop(0, KMAX)
    def _chunk_loop(k):
        chunk = k * NWORK + wid

        @pl.when(chunk < NCHUNK)
        def _():
            tok0 = chunk * CHUNK
            c1 = pltpu.async_copy(qid_i.at[pl.ds(tok0, 128)], qid_idx, sem_idx)
            c2 = pltpu.async_copy(qsrc_i.at[pl.ds(tok0, 128)], qsrc_idx,
                                  sem_idx)
            c3 = pltpu.async_copy(qw_i.at[pl.ds(tok0 * 8, 1024)], qw_idx,
                                  sem_idx)
            c4 = pltpu.async_copy(ii_i.at[pl.ds(tok0 * 3, 384)], ii_idx,
                                  sem_idx)
            c5 = pltpu.async_copy(ic_i.at[pl.ds(tok0 * 3, 384)], ic_idx,
                                  sem_idx)
            z = pltpu.async_copy(zeros_in,
                                 pooled_sh.at[pl.ds(sid * 128, 128)], sem_sa)
            c1.wait(); c2.wait(); c3.wait(); c4.wait(); c5.wait()
            gq = [pltpu.async_copy(qw_t.at[qw_idx.at[pl.ds(j * 128, 128)]],
                                   qw_rows.at[pl.ds(j * 128, 128)], sem_qw)
                  for j in range(8)]
            g1 = pltpu.async_copy(qid_t.at[qid_idx], qid_rows, sem_g)
            g2 = pltpu.async_copy(qsrc_t.at[qsrc_idx], qsrc_rows, sem_g)
            gi = [pltpu.async_copy(iid_t.at[ii_idx.at[pl.ds(j * 128, 128)]],
                                   ii_rows.at[pl.ds(j * 128, 128)], sem_g)
                  for j in range(3)]
            gc = [pltpu.async_copy(icat_t.at[ic_idx.at[pl.ds(j * 128, 128)]],
                                   ic_rows.at[pl.ds(j * 128, 128)], sem_g)
                  for j in range(3)]
            z.wait()
            for g in gq:
                g.wait()
            sa = [pltpu.async_copy(qw_rows.at[pl.ds(j * 128, 128)],
                                   pooled_sh.at[dst_idx.at[j]], sem_sa,
                                   add=True)
                  for j in range(8)]
            g1.wait(); g2.wait()
            for g in gi:
                g.wait()
            for g in gc:
                g.wait()
            o1 = pltpu.async_copy(
                qid_rows, qcat_o.at[pl.ds(tok0, CHUNK), pl.ds(0, D_ID)],
                sem_out)
            o2 = pltpu.async_copy(qsrc_rows, qsrc_o.at[pl.ds(tok0, CHUNK)],
                                  sem_out)
            o3 = pltpu.async_copy(
                ii_rows, item_o.at[pl.ds(chunk * 384, 384), pl.ds(0, D_ID)],
                sem_out)
            o4 = pltpu.async_copy(
                ic_rows,
                item_o.at[pl.ds(chunk * 384, 384), pl.ds(D_ID, D_CAT)],
                sem_out)
            for s in sa:
                s.wait()
            o5 = pltpu.async_copy(
                pooled_sh.at[pl.ds(sid * 128, 128)],
                qcat_o.at[pl.ds(tok0, CHUNK), pl.ds(D_ID, D_W)], sem_out)
            o1.wait(); o2.wait(); o3.wait(); o4.wait(); o5.wait()


def _sc_gather(qid2d, qsrc2d, qw2d, ii2d, ic2d,
               qid_emb, qword_emb, qsrc_emb, iid_emb, icat_emb):
    mesh = plsc.VectorSubcoreMesh(core_axis_name="c", subcore_axis_name="s",
                                  num_cores=2, num_subcores=16)
    f32 = jnp.float32
    out_type = (
        jax.ShapeDtypeStruct((T, 128), f32),
        jax.ShapeDtypeStruct((T, D_SRC), f32),
        jax.ShapeDtypeStruct((T * NC, 128), f32),
    )
    scratch = [
        pltpu.VMEM((128,), jnp.int32),
        pltpu.VMEM((128,), jnp.int32),
        pltpu.VMEM((1024,), jnp.int32),
        pltpu.VMEM((384,), jnp.int32),
        pltpu.VMEM((384,), jnp.int32),
        pltpu.VMEM((8, 128), jnp.int32),
        pltpu.VMEM((128, D_ID), f32),
        pltpu.VMEM((128, D_SRC), f32),
        pltpu.VMEM((1024, D_W), f32),
        pltpu.VMEM((384, D_ID), f32),
        pltpu.VMEM((384, D_CAT), f32),
        pltpu.VMEM_SHARED((16 * 128, D_W), f32),
        pltpu.SemaphoreType.DMA,
        pltpu.SemaphoreType.DMA,
        pltpu.SemaphoreType.DMA,
        pltpu.SemaphoreType.DMA,
        pltpu.SemaphoreType.DMA,
    ]
    fn = pl.kernel(_sc_body, out_type=out_type, mesh=mesh,
                   scratch_types=scratch,
                   compiler_params=pltpu.CompilerParams(
                       use_tc_tiling_on_sc=False))
    zeros_in = jnp.zeros((CHUNK, D_W), f32)
    return fn(qid2d, qsrc2d, qw2d, ii2d, ic2d,
              qid_emb, qword_emb, qsrc_emb, iid_emb, icat_emb, zeros_in)


QBLK = 1024
IBLK = QBLK * NC
GRID = T // QBLK


def _tc_body(qcat_r, words_r, emb0p_r, cmask_r, wq_r, bq_r,
             item_r, wi_r, bi_r, iidm_r,
             qout_r, iout_r, mask_r):
    words = words_r[...]
    wc = jnp.sum((words != 0).astype(jnp.float32), axis=1, keepdims=True)
    rdenom = 1.0 / jnp.maximum(wc, 1.0)
    scale = 1.0 + (rdenom - 1.0) * cmask_r[...]
    corrected = qcat_r[...] * scale - ((NW - wc) * rdenom) * emb0p_r[...]
    qout_r[...] = (
        jnp.dot(corrected, wq_r[...], preferred_element_type=jnp.float32)
        + bq_r[...]
    )
    iout_r[...] = (
        jnp.dot(item_r[..., :ITEM_SIZE], wi_r[...],
                preferred_element_type=jnp.float32)
        + bi_r[...]
    )
    mask_r[...] = iidm_r[...] != 0


def _tc_project(qcat, words2, emb0p, cmask, wq, bq, item, wi, bi, iidm3d):
    f32 = jnp.float32
    full = lambda shape: pl.BlockSpec(shape, lambda g: (0, 0))
    return pl.pallas_call(
        _tc_body,
        grid=(GRID,),
        in_specs=[
            pl.BlockSpec((QBLK, 128), lambda g: (g, 0)),
            pl.BlockSpec((QBLK, NW), lambda g: (g, 0)),
            full((1, 128)),
            full((1, 128)),
            full((128, ITEM_SIZE)),
            full((1, ITEM_SIZE)),
            pl.BlockSpec((IBLK, 128), lambda g: (g, 0)),
            full((ITEM_SIZE, ITEM_SIZE)),
            full((1, ITEM_SIZE)),
            pl.BlockSpec((1, 1, IBLK), lambda g: (g, 0, 0)),
        ],
        out_specs=[
            pl.BlockSpec((QBLK, ITEM_SIZE), lambda g: (g, 0)),
            pl.BlockSpec((IBLK, ITEM_SIZE), lambda g: (g, 0)),
            pl.BlockSpec((1, 1, IBLK), lambda g: (g, 0, 0)),
        ],
        out_shape=[
            jax.ShapeDtypeStruct((T, ITEM_SIZE), f32),
            jax.ShapeDtypeStruct((T * NC, ITEM_SIZE), f32),
            jax.ShapeDtypeStruct((GRID, 1, IBLK), jnp.bool_),
        ],
    )(qcat, words2, emb0p, cmask, wq, bq, item, wi, bi, iidm3d)


def kernel(query_id, search_source, click_item_id, click_item_category,
           query_words, item_id_emb, item_cat_emb, query_id_emb, qsrc_emb,
           qword_emb, W_q, b_q, W_i, b_i):
    i32 = jnp.int32
    f32 = jnp.float32
    qid2d = query_id.astype(i32).reshape(T)
    qsrc2d = search_source.astype(i32).reshape(T)
    qw2d = query_words.astype(i32).reshape(T * NW)
    ii2d = click_item_id.astype(i32).reshape(T * NC)
    ic2d = click_item_category.astype(i32).reshape(T * NC)

    qcat, qsrc_rows, item_rows = _sc_gather(
        qid2d, qsrc2d, qw2d, ii2d, ic2d,
        query_id_emb, qword_emb, qsrc_emb, item_id_emb, item_cat_emb)

    words2 = query_words.astype(i32).reshape(T, NW)
    iidm3d = click_item_id.astype(i32).reshape(GRID, 1, IBLK)
    emb0p = jnp.concatenate(
        [jnp.zeros((1, D_ID), f32), qword_emb[0:1, :]], axis=1)
    cmask = jnp.concatenate(
        [jnp.zeros((1, D_ID), f32), jnp.ones((1, D_W), f32)], axis=1)
    query_emb, item_out, mask3d = _tc_project(
        qcat, words2, emb0p, cmask, W_q, b_q.reshape(1, ITEM_SIZE),
        item_rows, W_i, b_i.reshape(1, ITEM_SIZE), iidm3d)

    return (query_emb.reshape(B, L, ITEM_SIZE),
            qsrc_rows.reshape(B, L, D_SRC),
            item_out.reshape(B, L, NC, ITEM_SIZE),
            mask3d.reshape(B, L, NC))


# speedup vs baseline: 3.5869x; 1.0001x over previous
import functools

import jax
import jax.numpy as jnp
from jax import lax
from jax.experimental import pallas as pl
from jax.experimental.pallas import tpu as pltpu
from jax.experimental.pallas import tpu_sc as plsc

B, L, NW, NC = 1024, 50, 8, 3
T = B * L
CHUNK = 128
NCHUNK = T // CHUNK
NWORK = 32
KMAX = -(-NCHUNK // NWORK)
D_ID, D_CAT, D_SRC, D_W = 64, 32, 16, 64
ITEM_SIZE = 96


def _sc_body(qid_i, qsrc_i, qw_i, ii_i, ic_i,
             qid_t, qw_t, qsrc_t, iid_t, icat_t, zeros_in,
             qcat_o, qsrc_o, item_o,
             qid_idx, qsrc_idx, qw_idx, ii_idx, ic_idx, dst_idx,
             qid_rows, qsrc_rows, qw_rows, ii_rows, ic_rows,
             pooled_sh, sem_idx, sem_qw, sem_g, sem_sa,
             sem_o1, sem_o2, sem_o3, sem_o4, sem_o5):
    cid = lax.axis_index("c")
    sid = lax.axis_index("s")
    wid = sid * 2 + cid

    lanes = lax.broadcasted_iota(jnp.int32, (16,), 0)

    for j in range(8):
        for v in range(8):
            vals = sid * 128 + 16 * j + 2 * v + (lanes >> 3)
            dst_idx[j, pl.ds(16 * v, 16)] = vals

    @pl.loop(0, KMAX)
    def _chunk_loop(k):
        chunk = k * NWORK + wid

        @pl.when(chunk < NCHUNK)
        def _():
            tok0 = chunk * CHUNK

            @pl.when(k > 0)
            def _drain():
                pltpu.make_async_copy(
                    pooled_sh.at[pl.ds(sid * 128, 128)],
                    qcat_o.at[pl.ds(tok0, CHUNK), pl.ds(D_ID, D_W)],
                    sem_o5).wait()
                pltpu.make_async_copy(
                    qid_rows, qcat_o.at[pl.ds(tok0, CHUNK), pl.ds(0, D_ID)],
                    sem_o1).wait()
                pltpu.make_async_copy(
                    qsrc_rows, qsrc_o.at[pl.ds(tok0, CHUNK)], sem_o2).wait()
                pltpu.make_async_copy(
                    ii_rows,
                    item_o.at[pl.ds(chunk * 384, 384), pl.ds(0, D_ID)],
                    sem_o3).wait()
                pltpu.make_async_copy(
                    ic_rows,
                    item_o.at[pl.ds(chunk * 384, 384), pl.ds(D_ID, D_CAT)],
                    sem_o4).wait()

            c1 = pltpu.async_copy(qid_i.at[pl.ds(tok0, 128)], qid_idx, sem_idx)
            c2 = pltpu.async_copy(qsrc_i.at[pl.ds(tok0, 128)], qsrc_idx,
                                  sem_idx)
            c3 = pltpu.async_copy(qw_i.at[pl.ds(tok0 * 8, 1024)], qw_idx,
                                  sem_idx)
            c4 = pltpu.async_copy(ii_i.at[pl.ds(tok0 * 3, 384)], ii_idx,
                                  sem_idx)
            c5 = pltpu.async_copy(ic_i.at[pl.ds(tok0 * 3, 384)], ic_idx,
                                  sem_idx)
            z = pltpu.async_copy(zeros_in,
                                 pooled_sh.at[pl.ds(sid * 128, 128)], sem_sa)
            c1.wait(); c2.wait(); c3.wait(); c4.wait(); c5.wait()
            gq = [pltpu.async_copy(qw_t.at[qw_idx.at[pl.ds(j * 128, 128)]],
                                   qw_rows.at[pl.ds(j * 128, 128)], sem_qw)
                  for j in range(8)]
            g1 = pltpu.async_copy(qid_t.at[qid_idx], qid_rows, sem_g)
            g2 = pltpu.async_copy(qsrc_t.at[qsrc_idx], qsrc_rows, sem_g)
            gi = [pltpu.async_copy(iid_t.at[ii_idx.at[pl.ds(j * 128, 128)]],
                                   ii_rows.at[pl.ds(j * 128, 128)], sem_g)
                  for j in range(3)]
            gc = [pltpu.async_copy(icat_t.at[ic_idx.at[pl.ds(j * 128, 128)]],
                                   ic_rows.at[pl.ds(j * 128, 128)], sem_g)
                  for j in range(3)]
            z.wait()
            for g in gq:
                g.wait()
            sa = [pltpu.async_copy(qw_rows.at[pl.ds(j * 128, 128)],
                                   pooled_sh.at[dst_idx.at[j]], sem_sa,
                                   add=True)
                  for j in range(8)]
            g1.wait(); g2.wait()
            for g in gi:
                g.wait()
            for g in gc:
                g.wait()
            pltpu.async_copy(
                qid_rows, qcat_o.at[pl.ds(tok0, CHUNK), pl.ds(0, D_ID)],
                sem_o1)
            pltpu.async_copy(qsrc_rows, qsrc_o.at[pl.ds(tok0, CHUNK)],
                             sem_o2)
            pltpu.async_copy(
                ii_rows, item_o.at[pl.ds(chunk * 384, 384), pl.ds(0, D_ID)],
                sem_o3)
            pltpu.async_copy(
                ic_rows,
                item_o.at[pl.ds(chunk * 384, 384), pl.ds(D_ID, D_CAT)],
                sem_o4)
            for s in sa:
                s.wait()
            pltpu.async_copy(
                pooled_sh.at[pl.ds(sid * 128, 128)],
                qcat_o.at[pl.ds(tok0, CHUNK), pl.ds(D_ID, D_W)], sem_o5)

    pltpu.make_async_copy(
        qid_rows, qcat_o.at[pl.ds(0, CHUNK), pl.ds(0, D_ID)], sem_o1).wait()
    pltpu.make_async_copy(
        qsrc_rows, qsrc_o.at[pl.ds(0, CHUNK)], sem_o2).wait()
    pltpu.make_async_copy(
        ii_rows, item_o.at[pl.ds(0, 384), pl.ds(0, D_ID)], sem_o3).wait()
    pltpu.make_async_copy(
        ic_rows, item_o.at[pl.ds(0, 384), pl.ds(D_ID, D_CAT)], sem_o4).wait()
    pltpu.make_async_copy(
        pooled_sh.at[pl.ds(sid * 128, 128)],
        qcat_o.at[pl.ds(0, CHUNK), pl.ds(D_ID, D_W)], sem_o5).wait()


def _sc_gather(qid2d, qsrc2d, qw2d, ii2d, ic2d,
               qid_emb, qword_emb, qsrc_emb, iid_emb, icat_emb):
    mesh = plsc.VectorSubcoreMesh(core_axis_name="c", subcore_axis_name="s",
                                  num_cores=2, num_subcores=16)
    f32 = jnp.float32
    out_type = (
        jax.ShapeDtypeStruct((T, 128), f32),
        jax.ShapeDtypeStruct((T, D_SRC), f32),
        jax.ShapeDtypeStruct((T * NC, 128), f32),
    )
    scratch = [
        pltpu.VMEM((128,), jnp.int32),
        pltpu.VMEM((128,), jnp.int32),
        pltpu.VMEM((1024,), jnp.int32),
        pltpu.VMEM((384,), jnp.int32),
        pltpu.VMEM((384,), jnp.int32),
        pltpu.VMEM((8, 128), jnp.int32),
        pltpu.VMEM((128, D_ID), f32),
        pltpu.VMEM((128, D_SRC), f32),
        pltpu.VMEM((1024, D_W), f32),
        pltpu.VMEM((384, D_ID), f32),
        pltpu.VMEM((384, D_CAT), f32),
        pltpu.VMEM_SHARED((16 * 128, D_W), f32),
    ] + [pltpu.SemaphoreType.DMA] * 9
    fn = pl.kernel(_sc_body, out_type=out_type, mesh=mesh,
                   scratch_types=scratch,
                   compiler_params=pltpu.CompilerParams(
                       use_tc_tiling_on_sc=False))
    zeros_in = jnp.zeros((CHUNK, D_W), f32)
    return fn(qid2d, qsrc2d, qw2d, ii2d, ic2d,
              qid_emb, qword_emb, qsrc_emb, iid_emb, icat_emb, zeros_in)


QBLK = 1024
IBLK = QBLK * NC
GRID = T // QBLK


def _tc_body(qcat_r, words_r, emb0p_r, cmask_r, wq_r, bq_r,
             item_r, wi_r, bi_r, iidm_r,
             qout_r, iout_r, mask_r):
    words = words_r[...]
    wc = jnp.sum((words != 0).astype(jnp.float32), axis=1, keepdims=True)
    rdenom = 1.0 / jnp.maximum(wc, 1.0)
    scale = 1.0 + (rdenom - 1.0) * cmask_r[...]
    corrected = qcat_r[...] * scale - ((NW - wc) * rdenom) * emb0p_r[...]
    qout_r[...] = (
        jnp.dot(corrected, wq_r[...], preferred_element_type=jnp.float32)
        + bq_r[...]
    )
    iout_r[...] = (
        jnp.dot(item_r[..., :ITEM_SIZE], wi_r[...],
                preferred_element_type=jnp.float32)
        + bi_r[...]
    )
    mask_r[...] = iidm_r[...] != 0


def _tc_project(qcat, words2, emb0p, cmask, wq, bq, item, wi, bi, iidm3d):
    f32 = jnp.float32
    full = lambda shape: pl.BlockSpec(shape, lambda g: (0, 0))
    return pl.pallas_call(
        _tc_body,
        grid=(GRID,),
        in_specs=[
            pl.BlockSpec((QBLK, 128), lambda g: (g, 0)),
            pl.BlockSpec((QBLK, NW), lambda g: (g, 0)),
            full((1, 128)),
            full((1, 128)),
            full((128, ITEM_SIZE)),
            full((1, ITEM_SIZE)),
            pl.BlockSpec((IBLK, 128), lambda g: (g, 0)),
            full((ITEM_SIZE, ITEM_SIZE)),
            full((1, ITEM_SIZE)),
            pl.BlockSpec((1, 1, IBLK), lambda g: (g, 0, 0)),
        ],
        out_specs=[
            pl.BlockSpec((QBLK, ITEM_SIZE), lambda g: (g, 0)),
            pl.BlockSpec((IBLK, ITEM_SIZE), lambda g: (g, 0)),
            pl.BlockSpec((1, 1, IBLK), lambda g: (g, 0, 0)),
        ],
        out_shape=[
            jax.ShapeDtypeStruct((T, ITEM_SIZE), f32),
            jax.ShapeDtypeStruct((T * NC, ITEM_SIZE), f32),
            jax.ShapeDtypeStruct((GRID, 1, IBLK), jnp.bool_),
        ],
    )(qcat, words2, emb0p, cmask, wq, bq, item, wi, bi, iidm3d)


def kernel(query_id, search_source, click_item_id, click_item_category,
           query_words, item_id_emb, item_cat_emb, query_id_emb, qsrc_emb,
           qword_emb, W_q, b_q, W_i, b_i):
    i32 = jnp.int32
    f32 = jnp.float32
    qid2d = query_id.astype(i32).reshape(T)
    qsrc2d = search_source.astype(i32).reshape(T)
    qw2d = query_words.astype(i32).reshape(T * NW)
    ii2d = click_item_id.astype(i32).reshape(T * NC)
    ic2d = click_item_category.astype(i32).reshape(T * NC)

    qcat, qsrc_rows, item_rows = _sc_gather(
        qid2d, qsrc2d, qw2d, ii2d, ic2d,
        query_id_emb, qword_emb, qsrc_emb, item_id_emb, item_cat_emb)

    words2 = query_words.astype(i32).reshape(T, NW)
    iidm3d = click_item_id.astype(i32).reshape(GRID, 1, IBLK)
    emb0p = jnp.concatenate(
        [jnp.zeros((1, D_ID), f32), qword_emb[0:1, :]], axis=1)
    cmask = jnp.concatenate(
        [jnp.zeros((1, D_ID), f32), jnp.ones((1, D_W), f32)], axis=1)
    query_emb, item_out, mask3d = _tc_project(
        qcat, words2, emb0p, cmask, W_q, b_q.reshape(1, ITEM_SIZE),
        item_rows, W_i, b_i.reshape(1, ITEM_SIZE), iidm3d)

    return (query_emb.reshape(B, L, ITEM_SIZE),
            qsrc_rows.reshape(B, L, D_SRC),
            item_out.reshape(B, L, NC, ITEM_SIZE),
            mask3d.reshape(B, L, NC))


# speedup vs baseline: 4.5757x; 1.2757x over previous
import functools

import jax
import jax.numpy as jnp
from jax import lax
from jax.experimental import pallas as pl
from jax.experimental.pallas import tpu as pltpu
from jax.experimental.pallas import tpu_sc as plsc

B, L, NW, NC = 1024, 50, 8, 3
T = B * L
CHUNK = 128
NCHUNK = T // CHUNK
NWORK = 32
KMAX = -(-NCHUNK // NWORK)
D_ID, D_CAT, D_SRC, D_W = 64, 32, 16, 64
ITEM_SIZE = 96


def _sc_body(qid_i, qw_i, ii_i, ic_i,
             qid_t, qw_t, iid_t, icat_t, zeros_in,
             qcat_o, item_o,
             qid_idx, qw_idx, ii_idx, ic_idx, dst_idx,
             qid_rows, qw_rows, ii_rows, ic_rows,
             pooled_sh, sem_idx, sem_qw, sem_g, sem_sa,
             sem_o1, sem_o3, sem_o4, sem_o5):
    cid = lax.axis_index("c")
    sid = lax.axis_index("s")
    wid = sid * 2 + cid

    lanes = lax.broadcasted_iota(jnp.int32, (16,), 0)

    for j in range(8):
        for v in range(8):
            vals = sid * 128 + 16 * j + 2 * v + (lanes >> 3)
            dst_idx[j, pl.ds(16 * v, 16)] = vals

    @pl.loop(0, KMAX)
    def _chunk_loop(k):
        chunk = k * NWORK + wid

        @pl.when(chunk < NCHUNK)
        def _():
            tok0 = chunk * CHUNK

            @pl.when(k > 0)
            def _drain():
                pltpu.make_async_copy(
                    pooled_sh.at[pl.ds(sid * 128, 128)],
                    qcat_o.at[pl.ds(tok0, CHUNK), pl.ds(D_ID, D_W)],
                    sem_o5).wait()
                pltpu.make_async_copy(
                    qid_rows, qcat_o.at[pl.ds(tok0, CHUNK), pl.ds(0, D_ID)],
                    sem_o1).wait()
                pltpu.make_async_copy(
                    ii_rows,
                    item_o.at[pl.ds(chunk * 384, 384), pl.ds(0, D_ID)],
                    sem_o3).wait()
                pltpu.make_async_copy(
                    ic_rows,
                    item_o.at[pl.ds(chunk * 384, 384), pl.ds(D_ID, D_CAT)],
                    sem_o4).wait()

            c1 = pltpu.async_copy(qid_i.at[pl.ds(tok0, 128)], qid_idx, sem_idx)
            c3 = pltpu.async_copy(qw_i.at[pl.ds(tok0 * 8, 1024)], qw_idx,
                                  sem_idx)
            c4 = pltpu.async_copy(ii_i.at[pl.ds(tok0 * 3, 384)], ii_idx,
                                  sem_idx)
            c5 = pltpu.async_copy(ic_i.at[pl.ds(tok0 * 3, 384)], ic_idx,
                                  sem_idx)
            z = pltpu.async_copy(zeros_in,
                                 pooled_sh.at[pl.ds(sid * 128, 128)], sem_sa)
            c1.wait(); c3.wait(); c4.wait(); c5.wait()
            gq = [pltpu.async_copy(qw_t.at[qw_idx.at[pl.ds(j * 128, 128)]],
                                   qw_rows.at[pl.ds(j * 128, 128)], sem_qw)
                  for j in range(8)]
            g1 = pltpu.async_copy(qid_t.at[qid_idx], qid_rows, sem_g)
            gi = [pltpu.async_copy(iid_t.at[ii_idx.at[pl.ds(j * 128, 128)]],
                                   ii_rows.at[pl.ds(j * 128, 128)], sem_g)
                  for j in range(3)]
            gc = [pltpu.async_copy(icat_t.at[ic_idx.at[pl.ds(j * 128, 128)]],
                                   ic_rows.at[pl.ds(j * 128, 128)], sem_g)
                  for j in range(3)]
            z.wait()
            for g in gq:
                g.wait()
            sa = [pltpu.async_copy(qw_rows.at[pl.ds(j * 128, 128)],
                                   pooled_sh.at[dst_idx.at[j]], sem_sa,
                                   add=True)
                  for j in range(8)]
            g1.wait()
            for g in gi:
                g.wait()
            for g in gc:
                g.wait()
            pltpu.async_copy(
                qid_rows, qcat_o.at[pl.ds(tok0, CHUNK), pl.ds(0, D_ID)],
                sem_o1)
            pltpu.async_copy(
                ii_rows, item_o.at[pl.ds(chunk * 384, 384), pl.ds(0, D_ID)],
                sem_o3)
            pltpu.async_copy(
                ic_rows,
                item_o.at[pl.ds(chunk * 384, 384), pl.ds(D_ID, D_CAT)],
                sem_o4)
            for s in sa:
                s.wait()
            pltpu.async_copy(
                pooled_sh.at[pl.ds(sid * 128, 128)],
                qcat_o.at[pl.ds(tok0, CHUNK), pl.ds(D_ID, D_W)], sem_o5)

    pltpu.make_async_copy(
        qid_rows, qcat_o.at[pl.ds(0, CHUNK), pl.ds(0, D_ID)], sem_o1).wait()
    pltpu.make_async_copy(
        ii_rows, item_o.at[pl.ds(0, 384), pl.ds(0, D_ID)], sem_o3).wait()
    pltpu.make_async_copy(
        ic_rows, item_o.at[pl.ds(0, 384), pl.ds(D_ID, D_CAT)], sem_o4).wait()
    pltpu.make_async_copy(
        pooled_sh.at[pl.ds(sid * 128, 128)],
        qcat_o.at[pl.ds(0, CHUNK), pl.ds(D_ID, D_W)], sem_o5).wait()


def _sc_gather(qid2d, qw2d, ii2d, ic2d, qid_emb, qword_emb, iid_emb,
               icat_emb):
    mesh = plsc.VectorSubcoreMesh(core_axis_name="c", subcore_axis_name="s",
                                  num_cores=2, num_subcores=16)
    f32 = jnp.float32
    out_type = (
        jax.ShapeDtypeStruct((T, 128), f32),
        jax.ShapeDtypeStruct((T * NC, 128), f32),
    )
    scratch = [
        pltpu.VMEM((128,), jnp.int32),
        pltpu.VMEM((1024,), jnp.int32),
        pltpu.VMEM((384,), jnp.int32),
        pltpu.VMEM((384,), jnp.int32),
        pltpu.VMEM((8, 128), jnp.int32),
        pltpu.VMEM((128, D_ID), f32),
        pltpu.VMEM((1024, D_W), f32),
        pltpu.VMEM((384, D_ID), f32),
        pltpu.VMEM((384, D_CAT), f32),
        pltpu.VMEM_SHARED((16 * 128, D_W), f32),
    ] + [pltpu.SemaphoreType.DMA] * 8
    fn = pl.kernel(_sc_body, out_type=out_type, mesh=mesh,
                   scratch_types=scratch,
                   compiler_params=pltpu.CompilerParams(
                       use_tc_tiling_on_sc=False))
    zeros_in = jnp.zeros((CHUNK, D_W), f32)
    return fn(qid2d, qw2d, ii2d, ic2d,
              qid_emb, qword_emb, iid_emb, icat_emb, zeros_in)


BB = 16
GRID = B // BB
QBLK = BB * L
IBLK = QBLK * NC


def _tc_body(qcat_r, ids_r, emb0p_r, cmask_r, wq_r, bq_r, qsrct_r,
             item_r, wi_r, bi_r, iidm_r,
             qout_r, qsrc_r, iout_r, mask_r):
    ids = ids_r[...]
    words = ids[:, :NW]
    wc = jnp.sum((words != 0).astype(jnp.float32), axis=1, keepdims=True)
    rdenom = 1.0 / jnp.maximum(wc, 1.0)
    scale = 1.0 + (rdenom - 1.0) * cmask_r[...]
    corrected = qcat_r[...] * scale - ((NW - wc) * rdenom) * emb0p_r[...]
    query = (
        jnp.dot(corrected, wq_r[...], preferred_element_type=jnp.float32)
        + bq_r[...]
    )
    sid = ids[:, NW:NW + 1]
    onehot = (sid == jax.lax.broadcasted_iota(jnp.int32, (1, 16), 1)
              ).astype(jnp.float32)
    qsrc = jnp.dot(onehot, qsrct_r[...], preferred_element_type=jnp.float32)
    for b in range(BB):
        qout_r[b] = query[b * L:(b + 1) * L, :]
        qsrc_r[b] = qsrc[b * L:(b + 1) * L, :]
    iout_r[...] = (
        jnp.dot(item_r[..., :ITEM_SIZE], wi_r[...],
                preferred_element_type=jnp.float32)
        + bi_r[...]
    )
    mask_r[...] = iidm_r[...] != 0


def _tc_project(qcat, ids16, emb0p, cmask, wq, bq, qsrct, item, wi, bi,
                iidm3d):
    f32 = jnp.float32
    full = lambda shape: pl.BlockSpec(shape, lambda g: tuple(0 for _ in shape))
    return pl.pallas_call(
        _tc_body,
        grid=(GRID,),
        in_specs=[
            pl.BlockSpec((QBLK, 128), lambda g: (g, 0)),
            pl.BlockSpec((QBLK, 16), lambda g: (g, 0)),
            full((1, 128)),
            full((1, 128)),
            full((128, ITEM_SIZE)),
            full((1, ITEM_SIZE)),
            full((16, D_SRC)),
            pl.BlockSpec((IBLK, 128), lambda g: (g, 0)),
            full((ITEM_SIZE, ITEM_SIZE)),
            full((1, ITEM_SIZE)),
            pl.BlockSpec((1, 1, IBLK), lambda g: (g, 0, 0)),
        ],
        out_specs=[
            pl.BlockSpec((BB, L, ITEM_SIZE), lambda g: (g, 0, 0)),
            pl.BlockSpec((BB, L, D_SRC), lambda g: (g, 0, 0)),
            pl.BlockSpec((IBLK, ITEM_SIZE), lambda g: (g, 0)),
            pl.BlockSpec((1, 1, IBLK), lambda g: (g, 0, 0)),
        ],
        out_shape=[
            jax.ShapeDtypeStruct((B, L, ITEM_SIZE), f32),
            jax.ShapeDtypeStruct((B, L, D_SRC), f32),
            jax.ShapeDtypeStruct((T * NC, ITEM_SIZE), f32),
            jax.ShapeDtypeStruct((GRID, 1, IBLK), jnp.bool_),
        ],
    )(qcat, ids16, emb0p, cmask, wq, bq, qsrct, item, wi, bi, iidm3d)


def kernel(query_id, search_source, click_item_id, click_item_category,
           query_words, item_id_emb, item_cat_emb, query_id_emb, qsrc_emb,
           qword_emb, W_q, b_q, W_i, b_i):
    i32 = jnp.int32
    f32 = jnp.float32
    qid2d = query_id.astype(i32).reshape(T)
    qw2d = query_words.astype(i32).reshape(T * NW)
    ii2d = click_item_id.astype(i32).reshape(T * NC)
    ic2d = click_item_category.astype(i32).reshape(T * NC)

    qcat, item_rows = _sc_gather(
        qid2d, qw2d, ii2d, ic2d,
        query_id_emb, qword_emb, item_id_emb, item_cat_emb)

    ids16 = jnp.concatenate(
        [query_words.astype(i32).reshape(T, NW),
         search_source.astype(i32).reshape(T, 1),
         jnp.zeros((T, 16 - NW - 1), i32)], axis=1)
    iidm3d = click_item_id.astype(i32).reshape(GRID, 1, IBLK)
    emb0p = jnp.concatenate(
        [jnp.zeros((1, D_ID), f32), qword_emb[0:1, :]], axis=1)
    cmask = jnp.concatenate(
        [jnp.zeros((1, D_ID), f32), jnp.ones((1, D_W), f32)], axis=1)
    qsrct = jnp.concatenate(
        [qsrc_emb, jnp.zeros((16 - qsrc_emb.shape[0], D_SRC), f32)], axis=0)
    query_emb, q_src_e, item_out, mask3d = _tc_project(
        qcat, ids16, emb0p, cmask, W_q, b_q.reshape(1, ITEM_SIZE), qsrct,
        item_rows, W_i, b_i.reshape(1, ITEM_SIZE), iidm3d)

    return (query_emb,
            q_src_e,
            item_out.reshape(B, L, NC, ITEM_SIZE),
            mask3d.reshape(B, L, NC))


# speedup vs baseline: 4.6476x; 1.0157x over previous
import functools

import jax
import jax.numpy as jnp
from jax import lax
from jax.experimental import pallas as pl
from jax.experimental.pallas import tpu as pltpu
from jax.experimental.pallas import tpu_sc as plsc

B, L, NW, NC = 1024, 50, 8, 3
T = B * L
CHUNK = 128
NCHUNK = T // CHUNK
NWORK = 32
KMAX = -(-NCHUNK // NWORK)
D_ID, D_CAT, D_SRC, D_W = 64, 32, 16, 64
ITEM_SIZE = 96


def _sc_body(qid_i, qw_i, ii_i, ic_i,
             qid_t, qw_t, iid_t, icat_t,
             qcat_o, item_o,
             qid_idx, qw_idx, ii_idx, ic_idx, dst_idx,
             qid_rows, qw_rows, ii_rows, ic_rows, zeros_v,
             pooled_sh, sem_idx, sem_g, sem_sa,
             sem_o1, sem_o3, sem_o4, sem_o5, *sem_qw):
    cid = lax.axis_index("c")
    sid = lax.axis_index("s")
    wid = sid * 2 + cid

    lanes = lax.broadcasted_iota(jnp.int32, (16,), 0)

    zvec = jnp.zeros((16,), jnp.float32)

    @pl.loop(0, 128)
    def _zinit(r):
        for c in range(4):
            zeros_v[r, pl.ds(c * 16, 16)] = zvec

    for j in range(8):
        for v in range(8):
            vals = sid * 128 + 16 * j + 2 * v + (lanes >> 3)
            dst_idx[j, pl.ds(16 * v, 16)] = vals

    @pl.loop(0, KMAX)
    def _chunk_loop(k):
        chunk = k * NWORK + wid

        @pl.when(chunk < NCHUNK)
        def _():
            tok0 = chunk * CHUNK

            @pl.when(k > 0)
            def _drain():
                pltpu.make_async_copy(
                    pooled_sh.at[pl.ds(sid * 128, 128)],
                    qcat_o.at[pl.ds(tok0, CHUNK), pl.ds(D_ID, D_W)],
                    sem_o5).wait()
                pltpu.make_async_copy(
                    qid_rows, qcat_o.at[pl.ds(tok0, CHUNK), pl.ds(0, D_ID)],
                    sem_o1).wait()
                pltpu.make_async_copy(
                    ii_rows,
                    item_o.at[pl.ds(chunk * 384, 384), pl.ds(0, D_ID)],
                    sem_o3).wait()
                pltpu.make_async_copy(
                    ic_rows,
                    item_o.at[pl.ds(chunk * 384, 384), pl.ds(D_ID, D_CAT)],
                    sem_o4).wait()

            c1 = pltpu.async_copy(qid_i.at[pl.ds(tok0, 128)], qid_idx, sem_idx)
            c3 = pltpu.async_copy(qw_i.at[pl.ds(tok0 * 8, 1024)], qw_idx,
                                  sem_idx)
            c4 = pltpu.async_copy(ii_i.at[pl.ds(tok0 * 3, 384)], ii_idx,
                                  sem_idx)
            c5 = pltpu.async_copy(ic_i.at[pl.ds(tok0 * 3, 384)], ic_idx,
                                  sem_idx)
            z = pltpu.async_copy(zeros_v,
                                 pooled_sh.at[pl.ds(sid * 128, 128)], sem_sa)
            c1.wait(); c3.wait(); c4.wait(); c5.wait()
            gq = [pltpu.async_copy(qw_t.at[qw_idx.at[pl.ds(j * 128, 128)]],
                                   qw_rows.at[pl.ds(j * 128, 128)], sem_qw[j])
                  for j in range(8)]
            g1 = pltpu.async_copy(qid_t.at[qid_idx], qid_rows, sem_g)
            gi = [pltpu.async_copy(iid_t.at[ii_idx.at[pl.ds(j * 128, 128)]],
                                   ii_rows.at[pl.ds(j * 128, 128)], sem_g)
                  for j in range(3)]
            gc = [pltpu.async_copy(icat_t.at[ic_idx.at[pl.ds(j * 128, 128)]],
                                   ic_rows.at[pl.ds(j * 128, 128)], sem_g)
                  for j in range(3)]
            z.wait()
            sa = []
            for j in range(8):
                gq[j].wait()
                sa.append(pltpu.async_copy(qw_rows.at[pl.ds(j * 128, 128)],
                                           pooled_sh.at[dst_idx.at[j]],
                                           sem_sa, add=True))
            g1.wait()
            for g in gi:
                g.wait()
            for g in gc:
                g.wait()
            pltpu.async_copy(
                qid_rows, qcat_o.at[pl.ds(tok0, CHUNK), pl.ds(0, D_ID)],
                sem_o1)
            pltpu.async_copy(
                ii_rows, item_o.at[pl.ds(chunk * 384, 384), pl.ds(0, D_ID)],
                sem_o3)
            pltpu.async_copy(
                ic_rows,
                item_o.at[pl.ds(chunk * 384, 384), pl.ds(D_ID, D_CAT)],
                sem_o4)
            for s in sa:
                s.wait()
            pltpu.async_copy(
                pooled_sh.at[pl.ds(sid * 128, 128)],
                qcat_o.at[pl.ds(tok0, CHUNK), pl.ds(D_ID, D_W)], sem_o5)

    pltpu.make_async_copy(
        qid_rows, qcat_o.at[pl.ds(0, CHUNK), pl.ds(0, D_ID)], sem_o1).wait()
    pltpu.make_async_copy(
        ii_rows, item_o.at[pl.ds(0, 384), pl.ds(0, D_ID)], sem_o3).wait()
    pltpu.make_async_copy(
        ic_rows, item_o.at[pl.ds(0, 384), pl.ds(D_ID, D_CAT)], sem_o4).wait()
    pltpu.make_async_copy(
        pooled_sh.at[pl.ds(sid * 128, 128)],
        qcat_o.at[pl.ds(0, CHUNK), pl.ds(D_ID, D_W)], sem_o5).wait()


def _sc_gather(qid2d, qw2d, ii2d, ic2d, qid_emb, qword_emb, iid_emb,
               icat_emb):
    mesh = plsc.VectorSubcoreMesh(core_axis_name="c", subcore_axis_name="s",
                                  num_cores=2, num_subcores=16)
    f32 = jnp.float32
    out_type = (
        jax.ShapeDtypeStruct((T, 128), f32),
        jax.ShapeDtypeStruct((T * NC, 128), f32),
    )
    scratch = [
        pltpu.VMEM((128,), jnp.int32),
        pltpu.VMEM((1024,), jnp.int32),
        pltpu.VMEM((384,), jnp.int32),
        pltpu.VMEM((384,), jnp.int32),
        pltpu.VMEM((8, 128), jnp.int32),
        pltpu.VMEM((128, D_ID), f32),
        pltpu.VMEM((1024, D_W), f32),
        pltpu.VMEM((384, D_ID), f32),
        pltpu.VMEM((384, D_CAT), f32),
        pltpu.VMEM((128, D_W), f32),
        pltpu.VMEM_SHARED((16 * 128, D_W), f32),
    ] + [pltpu.SemaphoreType.DMA] * 15
    fn = pl.kernel(_sc_body, out_type=out_type, mesh=mesh,
                   scratch_types=scratch,
                   compiler_params=pltpu.CompilerParams(
                       use_tc_tiling_on_sc=False))
    return fn(qid2d, qw2d, ii2d, ic2d,
              qid_emb, qword_emb, iid_emb, icat_emb)


BB = 16
GRID = B // BB
QBLK = BB * L
IBLK = QBLK * NC


def _tc_body(qcat_r, ids_r, emb0p_r, cmask_r, wq_r, bq_r, qsrct_r,
             item_r, wi_r, bi_r, iidm_r,
             qout_r, qsrc_r, iout_r, mask_r):
    ids = ids_r[...]
    words = ids[:, :NW]
    wc = jnp.sum((words != 0).astype(jnp.float32), axis=1, keepdims=True)
    rdenom = 1.0 / jnp.maximum(wc, 1.0)
    scale = 1.0 + (rdenom - 1.0) * cmask_r[...]
    corrected = qcat_r[...] * scale - ((NW - wc) * rdenom) * emb0p_r[...]
    query = (
        jnp.dot(corrected, wq_r[...], preferred_element_type=jnp.float32)
        + bq_r[...]
    )
    sid = ids[:, NW:NW + 1]
    onehot = (sid == jax.lax.broadcasted_iota(jnp.int32, (1, 16), 1)
              ).astype(jnp.float32)
    qsrc = jnp.dot(onehot, qsrct_r[...], preferred_element_type=jnp.float32,
                   precision=jax.lax.Precision.HIGHEST)
    for b in range(BB):
        qout_r[b] = query[b * L:(b + 1) * L, :]
        qsrc_r[b] = qsrc[b * L:(b + 1) * L, :]
    iout_r[...] = (
        jnp.dot(item_r[..., :ITEM_SIZE], wi_r[...],
                preferred_element_type=jnp.float32)
        + bi_r[...]
    )
    mask_r[...] = iidm_r[...] != 0


def _tc_project(qcat, ids16, emb0p, cmask, wq, bq, qsrct, item, wi, bi,
                iidm3d):
    f32 = jnp.float32
    full = lambda shape: pl.BlockSpec(shape, lambda g: tuple(0 for _ in shape))
    return pl.pallas_call(
        _tc_body,
        grid=(GRID,),
        in_specs=[
            pl.BlockSpec((QBLK, 128), lambda g: (g, 0)),
            pl.BlockSpec((QBLK, 16), lambda g: (g, 0)),
            full((1, 128)),
            full((1, 128)),
            full((128, ITEM_SIZE)),
            full((1, ITEM_SIZE)),
            full((16, D_SRC)),
            pl.BlockSpec((IBLK, 128), lambda g: (g, 0)),
            full((ITEM_SIZE, ITEM_SIZE)),
            full((1, ITEM_SIZE)),
            pl.BlockSpec((1, 1, IBLK), lambda g: (g, 0, 0)),
        ],
        out_specs=[
            pl.BlockSpec((BB, L, ITEM_SIZE), lambda g: (g, 0, 0)),
            pl.BlockSpec((BB, L, D_SRC), lambda g: (g, 0, 0)),
            pl.BlockSpec((IBLK, ITEM_SIZE), lambda g: (g, 0)),
            pl.BlockSpec((1, 1, IBLK), lambda g: (g, 0, 0)),
        ],
        out_shape=[
            jax.ShapeDtypeStruct((B, L, ITEM_SIZE), f32),
            jax.ShapeDtypeStruct((B, L, D_SRC), f32),
            jax.ShapeDtypeStruct((T * NC, ITEM_SIZE), f32),
            jax.ShapeDtypeStruct((GRID, 1, IBLK), jnp.bool_),
        ],
    )(qcat, ids16, emb0p, cmask, wq, bq, qsrct, item, wi, bi, iidm3d)


def kernel(query_id, search_source, click_item_id, click_item_category,
           query_words, item_id_emb, item_cat_emb, query_id_emb, qsrc_emb,
           qword_emb, W_q, b_q, W_i, b_i):
    i32 = jnp.int32
    f32 = jnp.float32
    qid2d = query_id.astype(i32).reshape(T)
    qw2d = query_words.astype(i32).reshape(T * NW)
    ii2d = click_item_id.astype(i32).reshape(T * NC)
    ic2d = click_item_category.astype(i32).reshape(T * NC)

    qcat, item_rows = _sc_gather(
        qid2d, qw2d, ii2d, ic2d,
        query_id_emb, qword_emb, item_id_emb, item_cat_emb)

    ids16 = jnp.concatenate(
        [query_words.astype(i32).reshape(T, NW),
         search_source.astype(i32).reshape(T, 1),
         jnp.zeros((T, 16 - NW - 1), i32)], axis=1)
    iidm3d = click_item_id.astype(i32).reshape(GRID, 1, IBLK)
    emb0p = jnp.concatenate(
        [jnp.zeros((1, D_ID), f32), qword_emb[0:1, :]], axis=1)
    cmask = jnp.concatenate(
        [jnp.zeros((1, D_ID), f32), jnp.ones((1, D_W), f32)], axis=1)
    qsrct = jnp.concatenate(
        [qsrc_emb, jnp.zeros((16 - qsrc_emb.shape[0], D_SRC), f32)], axis=0)
    query_emb, q_src_e, item_out, mask3d = _tc_project(
        qcat, ids16, emb0p, cmask, W_q, b_q.reshape(1, ITEM_SIZE), qsrct,
        item_rows, W_i, b_i.reshape(1, ITEM_SIZE), iidm3d)

    return (query_emb,
            q_src_e,
            item_out.reshape(B, L, NC, ITEM_SIZE),
            mask3d.reshape(B, L, NC))


# speedup vs baseline: 4.7371x; 1.0192x over previous
import functools

import jax
import jax.numpy as jnp
from jax import lax
from jax.experimental import pallas as pl
from jax.experimental.pallas import tpu as pltpu
from jax.experimental.pallas import tpu_sc as plsc

B, L, NW, NC = 1024, 50, 8, 3
T = B * L
CHUNK = 128
NCHUNK = T // CHUNK
NWORK = 32
KMAX = -(-NCHUNK // NWORK)
D_ID, D_CAT, D_SRC, D_W = 64, 32, 16, 64
ITEM_SIZE = 96


def _sc_body(qid_i, qw_i, ii_i, ic_i,
             qid_t, qw_t, iid_t, icat_t,
             qcat_o, item_o,
             qid_idx, qw_idx, ii_idx, ic_idx, dst_idx,
             qid_rows, qw_rows, ii_rows, ic_rows, zeros_v,
             pooled_sh, sem_idx, sem_g, sem_sa,
             sem_o1, sem_o3, sem_o4, sem_o5, *sem_qw):
    cid = lax.axis_index("c")
    sid = lax.axis_index("s")
    wid = sid * 2 + cid

    lanes = lax.broadcasted_iota(jnp.int32, (16,), 0)

    zvec = jnp.zeros((16,), jnp.float32)

    @pl.loop(0, 128)
    def _zinit(r):
        for c in range(4):
            zeros_v[r, pl.ds(c * 16, 16)] = zvec

    for j in range(8):
        for v in range(8):
            vals = sid * 128 + 16 * j + 2 * v + (lanes >> 3)
            dst_idx[j, pl.ds(16 * v, 16)] = vals

    @pl.loop(0, KMAX)
    def _chunk_loop(k):
        chunk = k * NWORK + wid

        @pl.when(chunk < NCHUNK)
        def _():
            tok0 = chunk * CHUNK

            @pl.when(k > 0)
            def _drain():
                pltpu.make_async_copy(
                    pooled_sh.at[pl.ds(sid * 128, 128)],
                    qcat_o.at[pl.ds(tok0, CHUNK), pl.ds(D_ID, D_W)],
                    sem_o5).wait()
                pltpu.make_async_copy(
                    qid_rows, qcat_o.at[pl.ds(tok0, CHUNK), pl.ds(0, D_ID)],
                    sem_o1).wait()
                pltpu.make_async_copy(
                    ii_rows,
                    item_o.at[pl.ds(chunk * 384, 384), pl.ds(0, D_ID)],
                    sem_o3).wait()
                pltpu.make_async_copy(
                    ic_rows,
                    item_o.at[pl.ds(chunk * 384, 384), pl.ds(D_ID, D_CAT)],
                    sem_o4).wait()

            c1 = pltpu.async_copy(qid_i.at[pl.ds(tok0, 128)], qid_idx, sem_idx)
            c3 = pltpu.async_copy(qw_i.at[pl.ds(tok0 * 8, 1024)], qw_idx,
                                  sem_idx)
            c4 = pltpu.async_copy(ii_i.at[pl.ds(tok0 * 3, 384)], ii_idx,
                                  sem_idx)
            c5 = pltpu.async_copy(ic_i.at[pl.ds(tok0 * 3, 384)], ic_idx,
                                  sem_idx)
            z = pltpu.async_copy(zeros_v,
                                 pooled_sh.at[pl.ds(sid * 128, 128)], sem_sa)
            c1.wait(); c3.wait(); c4.wait(); c5.wait()
            gq = [pltpu.async_copy(qw_t.at[qw_idx.at[pl.ds(j * 128, 128)]],
                                   qw_rows.at[pl.ds(j * 128, 128)], sem_qw[j])
                  for j in range(8)]
            g1 = pltpu.async_copy(qid_t.at[qid_idx], qid_rows, sem_g)
            gi = [pltpu.async_copy(iid_t.at[ii_idx.at[pl.ds(j * 128, 128)]],
                                   ii_rows.at[pl.ds(j * 128, 128)], sem_g)
                  for j in range(3)]
            gc = [pltpu.async_copy(icat_t.at[ic_idx.at[pl.ds(j * 128, 128)]],
                                   ic_rows.at[pl.ds(j * 128, 128)], sem_g)
                  for j in range(3)]
            z.wait()
            sa = []
            for j in range(8):
                gq[j].wait()
                sa.append(pltpu.async_copy(qw_rows.at[pl.ds(j * 128, 128)],
                                           pooled_sh.at[dst_idx.at[j]],
                                           sem_sa, add=True))
            g1.wait()
            for g in gi:
                g.wait()
            for g in gc:
                g.wait()
            pltpu.async_copy(
                qid_rows, qcat_o.at[pl.ds(tok0, CHUNK), pl.ds(0, D_ID)],
                sem_o1)
            pltpu.async_copy(
                ii_rows, item_o.at[pl.ds(chunk * 384, 384), pl.ds(0, D_ID)],
                sem_o3)
            pltpu.async_copy(
                ic_rows,
                item_o.at[pl.ds(chunk * 384, 384), pl.ds(D_ID, D_CAT)],
                sem_o4)
            for s in sa:
                s.wait()
            pltpu.async_copy(
                pooled_sh.at[pl.ds(sid * 128, 128)],
                qcat_o.at[pl.ds(tok0, CHUNK), pl.ds(D_ID, D_W)], sem_o5)

    pltpu.make_async_copy(
        qid_rows, qcat_o.at[pl.ds(0, CHUNK), pl.ds(0, D_ID)], sem_o1).wait()
    pltpu.make_async_copy(
        ii_rows, item_o.at[pl.ds(0, 384), pl.ds(0, D_ID)], sem_o3).wait()
    pltpu.make_async_copy(
        ic_rows, item_o.at[pl.ds(0, 384), pl.ds(D_ID, D_CAT)], sem_o4).wait()
    pltpu.make_async_copy(
        pooled_sh.at[pl.ds(sid * 128, 128)],
        qcat_o.at[pl.ds(0, CHUNK), pl.ds(D_ID, D_W)], sem_o5).wait()


def _sc_gather(qid2d, qw2d, ii2d, ic2d, qid_emb, qword_emb, iid_emb,
               icat_emb):
    mesh = plsc.VectorSubcoreMesh(core_axis_name="c", subcore_axis_name="s",
                                  num_cores=2, num_subcores=16)
    f32 = jnp.float32
    out_type = (
        jax.ShapeDtypeStruct((T, 128), f32),
        jax.ShapeDtypeStruct((T * NC, 128), f32),
    )
    scratch = [
        pltpu.VMEM((128,), jnp.int32),
        pltpu.VMEM((1024,), jnp.int32),
        pltpu.VMEM((384,), jnp.int32),
        pltpu.VMEM((384,), jnp.int32),
        pltpu.VMEM((8, 128), jnp.int32),
        pltpu.VMEM((128, D_ID), f32),
        pltpu.VMEM((1024, D_W), f32),
        pltpu.VMEM((384, D_ID), f32),
        pltpu.VMEM((384, D_CAT), f32),
        pltpu.VMEM((128, D_W), f32),
        pltpu.VMEM_SHARED((16 * 128, D_W), f32),
    ] + [pltpu.SemaphoreType.DMA] * 15
    fn = pl.kernel(_sc_body, out_type=out_type, mesh=mesh,
                   scratch_types=scratch,
                   compiler_params=pltpu.CompilerParams(
                       use_tc_tiling_on_sc=False))
    return fn(qid2d, qw2d, ii2d, ic2d,
              qid_emb, qword_emb, iid_emb, icat_emb)


BB = 32
GRID = B // BB
QBLK = BB * L
IBLK = QBLK * NC


def _tc_body(qcat_r, ids_r, emb0p_r, cmask_r, wq_r, bq_r, qsrct_r,
             item_r, wi_r, bi_r, iidm_r,
             qout_r, qsrc_r, iout_r, mask_r):
    ids = ids_r[...]
    words = ids[:, :NW]
    wc = jnp.sum((words != 0).astype(jnp.float32), axis=1, keepdims=True)
    rdenom = 1.0 / jnp.maximum(wc, 1.0)
    scale = 1.0 + (rdenom - 1.0) * cmask_r[...]
    corrected = qcat_r[...] * scale - ((NW - wc) * rdenom) * emb0p_r[...]
    query = (
        jnp.dot(corrected, wq_r[...], preferred_element_type=jnp.float32)
        + bq_r[...]
    )
    sid = ids[:, NW:NW + 1]
    onehot = (sid == jax.lax.broadcasted_iota(jnp.int32, (1, 16), 1)
              ).astype(jnp.float32)
    qsrc = jnp.dot(onehot, qsrct_r[...], preferred_element_type=jnp.float32,
                   precision=jax.lax.Precision.HIGHEST)
    for b in range(BB):
        qout_r[b] = query[b * L:(b + 1) * L, :]
        qsrc_r[b] = qsrc[b * L:(b + 1) * L, :]
    iout_r[...] = (
        jnp.dot(item_r[..., :ITEM_SIZE], wi_r[...],
                preferred_element_type=jnp.float32)
        + bi_r[...]
    )
    mask_r[...] = iidm_r[...] != 0


def _tc_project(qcat, ids16, emb0p, cmask, wq, bq, qsrct, item, wi, bi,
                iidm3d):
    f32 = jnp.float32
    full = lambda shape: pl.BlockSpec(shape, lambda g: tuple(0 for _ in shape))
    return pl.pallas_call(
        _tc_body,
        grid=(GRID,),
        in_specs=[
            pl.BlockSpec((QBLK, 128), lambda g: (g, 0)),
            pl.BlockSpec((QBLK, 16), lambda g: (g, 0)),
            full((1, 128)),
            full((1, 128)),
            full((128, ITEM_SIZE)),
            full((1, ITEM_SIZE)),
            full((16, D_SRC)),
            pl.BlockSpec((IBLK, 128), lambda g: (g, 0)),
            full((ITEM_SIZE, ITEM_SIZE)),
            full((1, ITEM_SIZE)),
            pl.BlockSpec((1, 1, IBLK), lambda g: (g, 0, 0)),
        ],
        out_specs=[
            pl.BlockSpec((BB, L, ITEM_SIZE), lambda g: (g, 0, 0)),
            pl.BlockSpec((BB, L, D_SRC), lambda g: (g, 0, 0)),
            pl.BlockSpec((IBLK, ITEM_SIZE), lambda g: (g, 0)),
            pl.BlockSpec((1, 1, IBLK), lambda g: (g, 0, 0)),
        ],
        out_shape=[
            jax.ShapeDtypeStruct((B, L, ITEM_SIZE), f32),
            jax.ShapeDtypeStruct((B, L, D_SRC), f32),
            jax.ShapeDtypeStruct((T * NC, ITEM_SIZE), f32),
            jax.ShapeDtypeStruct((GRID, 1, IBLK), jnp.bool_),
        ],
    )(qcat, ids16, emb0p, cmask, wq, bq, qsrct, item, wi, bi, iidm3d)


def kernel(query_id, search_source, click_item_id, click_item_category,
           query_words, item_id_emb, item_cat_emb, query_id_emb, qsrc_emb,
           qword_emb, W_q, b_q, W_i, b_i):
    i32 = jnp.int32
    f32 = jnp.float32
    qid2d = query_id.astype(i32).reshape(T)
    qw2d = query_words.astype(i32).reshape(T * NW)
    ii2d = click_item_id.astype(i32).reshape(T * NC)
    ic2d = click_item_category.astype(i32).reshape(T * NC)

    qcat, item_rows = _sc_gather(
        qid2d, qw2d, ii2d, ic2d,
        query_id_emb, qword_emb, item_id_emb, item_cat_emb)

    ids16 = jnp.concatenate(
        [query_words.astype(i32).reshape(T, NW),
         search_source.astype(i32).reshape(T, 1),
         jnp.zeros((T, 16 - NW - 1), i32)], axis=1)
    iidm3d = click_item_id.astype(i32).reshape(GRID, 1, IBLK)
    emb0p = jnp.concatenate(
        [jnp.zeros((1, D_ID), f32), qword_emb[0:1, :]], axis=1)
    cmask = jnp.concatenate(
        [jnp.zeros((1, D_ID), f32), jnp.ones((1, D_W), f32)], axis=1)
    qsrct = jnp.concatenate(
        [qsrc_emb, jnp.zeros((16 - qsrc_emb.shape[0], D_SRC), f32)], axis=0)
    query_emb, q_src_e, item_out, mask3d = _tc_project(
        qcat, ids16, emb0p, cmask, W_q, b_q.reshape(1, ITEM_SIZE), qsrct,
        item_rows, W_i, b_i.reshape(1, ITEM_SIZE), iidm3d)

    return (query_emb,
            q_src_e,
            item_out.reshape(B, L, NC, ITEM_SIZE),
            mask3d.reshape(B, L, NC))


# speedup vs baseline: 4.9254x; 1.0398x over previous
import functools

import jax
import jax.numpy as jnp
from jax import lax
from jax.experimental import pallas as pl
from jax.experimental.pallas import tpu as pltpu
from jax.experimental.pallas import tpu_sc as plsc

B, L, NW, NC = 1024, 50, 8, 3
T = B * L
CHUNK = 128
NCHUNK = T // CHUNK
NWORK = 32
KMAX = -(-NCHUNK // NWORK)
D_ID, D_CAT, D_SRC, D_W = 64, 32, 16, 64
ITEM_SIZE = 96


def _sc_body(qid_i, qw_i, ii_i, ic_i,
             qid_t, qw_t, iid_t, icat_t,
             qcat_o, item_o,
             qid_idx, qw_idx, ii_idx, ic_idx, dst_idx,
             qid_rows, qw_rows, ii_rows, ic_rows, zeros_v,
             pooled_sh, sem_idx, sem_g, sem_sa,
             sem_o1, sem_o3, sem_o4, sem_o5, *sem_qw):
    cid = lax.axis_index("c")
    sid = lax.axis_index("s")
    wid = sid * 2 + cid

    lanes = lax.broadcasted_iota(jnp.int32, (16,), 0)

    zvec = jnp.zeros((16,), jnp.float32)

    @pl.loop(0, 128)
    def _zinit(r):
        for c in range(4):
            zeros_v[r, pl.ds(c * 16, 16)] = zvec

    for j in range(8):
        for v in range(8):
            vals = sid * 128 + 16 * j + 2 * v + (lanes >> 3)
            dst_idx[j, pl.ds(16 * v, 16)] = vals

    @pl.loop(0, KMAX)
    def _chunk_loop(k):
        chunk = k * NWORK + wid

        @pl.when(chunk < NCHUNK)
        def _():
            tok0 = chunk * CHUNK

            @pl.when(k > 0)
            def _drain():
                pltpu.make_async_copy(
                    pooled_sh.at[pl.ds(sid * 128, 128)],
                    qcat_o.at[pl.ds(tok0, CHUNK), pl.ds(D_ID, D_W)],
                    sem_o5).wait()
                pltpu.make_async_copy(
                    qid_rows, qcat_o.at[pl.ds(tok0, CHUNK), pl.ds(0, D_ID)],
                    sem_o1).wait()
                pltpu.make_async_copy(
                    ii_rows,
                    item_o.at[pl.ds(chunk * 384, 384), pl.ds(0, D_ID)],
                    sem_o3).wait()
                pltpu.make_async_copy(
                    ic_rows,
                    item_o.at[pl.ds(chunk * 384, 384), pl.ds(D_ID, D_CAT)],
                    sem_o4).wait()

            c1 = pltpu.async_copy(qid_i.at[pl.ds(tok0, 128)], qid_idx, sem_idx)
            c3 = pltpu.async_copy(qw_i.at[pl.ds(tok0 * 8, 1024)], qw_idx,
                                  sem_idx)
            c4 = pltpu.async_copy(ii_i.at[pl.ds(tok0 * 3, 384)], ii_idx,
                                  sem_idx)
            c5 = pltpu.async_copy(ic_i.at[pl.ds(tok0 * 3, 384)], ic_idx,
                                  sem_idx)
            z = pltpu.async_copy(zeros_v,
                                 pooled_sh.at[pl.ds(sid * 128, 128)], sem_sa)
            c1.wait(); c3.wait(); c4.wait(); c5.wait()
            gq = [pltpu.async_copy(qw_t.at[qw_idx.at[pl.ds(j * 128, 128)]],
                                   qw_rows.at[pl.ds(j * 128, 128)], sem_qw[j])
                  for j in range(8)]
            g1 = pltpu.async_copy(qid_t.at[qid_idx], qid_rows, sem_g)
            gi = [pltpu.async_copy(iid_t.at[ii_idx.at[pl.ds(j * 128, 128)]],
                                   ii_rows.at[pl.ds(j * 128, 128)], sem_g)
                  for j in range(3)]
            gc = [pltpu.async_copy(icat_t.at[ic_idx.at[pl.ds(j * 128, 128)]],
                                   ic_rows.at[pl.ds(j * 128, 128)], sem_g)
                  for j in range(3)]
            z.wait()
            sa = []
            for j in range(8):
                gq[j].wait()
                sa.append(pltpu.async_copy(qw_rows.at[pl.ds(j * 128, 128)],
                                           pooled_sh.at[dst_idx.at[j]],
                                           sem_sa, add=True))
            g1.wait()
            for g in gi:
                g.wait()
            for g in gc:
                g.wait()
            pltpu.async_copy(
                qid_rows, qcat_o.at[pl.ds(tok0, CHUNK), pl.ds(0, D_ID)],
                sem_o1)
            pltpu.async_copy(
                ii_rows, item_o.at[pl.ds(chunk * 384, 384), pl.ds(0, D_ID)],
                sem_o3)
            pltpu.async_copy(
                ic_rows,
                item_o.at[pl.ds(chunk * 384, 384), pl.ds(D_ID, D_CAT)],
                sem_o4)
            for s in sa:
                s.wait()
            pltpu.async_copy(
                pooled_sh.at[pl.ds(sid * 128, 128)],
                qcat_o.at[pl.ds(tok0, CHUNK), pl.ds(D_ID, D_W)], sem_o5)

    pltpu.make_async_copy(
        qid_rows, qcat_o.at[pl.ds(0, CHUNK), pl.ds(0, D_ID)], sem_o1).wait()
    pltpu.make_async_copy(
        ii_rows, item_o.at[pl.ds(0, 384), pl.ds(0, D_ID)], sem_o3).wait()
    pltpu.make_async_copy(
        ic_rows, item_o.at[pl.ds(0, 384), pl.ds(D_ID, D_CAT)], sem_o4).wait()
    pltpu.make_async_copy(
        pooled_sh.at[pl.ds(sid * 128, 128)],
        qcat_o.at[pl.ds(0, CHUNK), pl.ds(D_ID, D_W)], sem_o5).wait()


def _sc_gather(qid2d, qw2d, ii2d, ic2d, qid_emb, qword_emb, iid_emb,
               icat_emb):
    mesh = plsc.VectorSubcoreMesh(core_axis_name="c", subcore_axis_name="s",
                                  num_cores=2, num_subcores=16)
    f32 = jnp.float32
    out_type = (
        jax.ShapeDtypeStruct((T, 128), f32),
        jax.ShapeDtypeStruct((T * NC, 128), f32),
    )
    scratch = [
        pltpu.VMEM((128,), jnp.int32),
        pltpu.VMEM((1024,), jnp.int32),
        pltpu.VMEM((384,), jnp.int32),
        pltpu.VMEM((384,), jnp.int32),
        pltpu.VMEM((8, 128), jnp.int32),
        pltpu.VMEM((128, D_ID), f32),
        pltpu.VMEM((1024, D_W), f32),
        pltpu.VMEM((384, D_ID), f32),
        pltpu.VMEM((384, D_CAT), f32),
        pltpu.VMEM((128, D_W), f32),
        pltpu.VMEM_SHARED((16 * 128, D_W), f32),
    ] + [pltpu.SemaphoreType.DMA] * 15
    fn = pl.kernel(_sc_body, out_type=out_type, mesh=mesh,
                   scratch_types=scratch,
                   compiler_params=pltpu.CompilerParams(
                       use_tc_tiling_on_sc=False))
    return fn(qid2d, qw2d, ii2d, ic2d,
              qid_emb, qword_emb, iid_emb, icat_emb)


BB = 32
GRID = B // BB
QBLK = BB * L
IBLK = QBLK * NC


def _tc_body(qcat_r, ids_r, emb0p_r, cmask_r, wq_r, bq_r, qsrct_r,
             item_r, wi_r, bi_r, iidm_r,
             qout_r, qsrc_r, iout_r, mask_r):
    ids = ids_r[...]
    words = ids[:, :NW]
    wc = jnp.sum((words != 0).astype(jnp.float32), axis=1, keepdims=True)
    rdenom = 1.0 / jnp.maximum(wc, 1.0)
    scale = 1.0 + (rdenom - 1.0) * cmask_r[...]
    corrected = qcat_r[...] * scale - ((NW - wc) * rdenom) * emb0p_r[...]
    query = (
        jnp.dot(corrected, wq_r[...], preferred_element_type=jnp.float32)
        + bq_r[...]
    )
    sid = ids[:, NW:NW + 1]
    onehot = (sid == jax.lax.broadcasted_iota(jnp.int32, (1, 16), 1)
              ).astype(jnp.float32)
    qsrc = jnp.dot(onehot, qsrct_r[...], preferred_element_type=jnp.float32,
                   precision=jax.lax.Precision.HIGHEST)
    iout = (
        jnp.dot(item_r[..., :ITEM_SIZE], wi_r[...],
                preferred_element_type=jnp.float32)
        + bi_r[...]
    )
    for b in range(BB):
        qout_r[b] = query[b * L:(b + 1) * L, :]
        qsrc_r[b] = qsrc[b * L:(b + 1) * L, :]
        iout_r[b] = iout[b * L * NC:(b + 1) * L * NC, :].reshape(
            L, NC, ITEM_SIZE)
    mask_r[...] = iidm_r[...] != 0


def _tc_project(qcat, ids16, emb0p, cmask, wq, bq, qsrct, item, wi, bi,
                iidm3d):
    f32 = jnp.float32
    full = lambda shape: pl.BlockSpec(shape, lambda g: tuple(0 for _ in shape))
    return pl.pallas_call(
        _tc_body,
        grid=(GRID,),
        in_specs=[
            pl.BlockSpec((QBLK, 128), lambda g: (g, 0)),
            pl.BlockSpec((QBLK, 16), lambda g: (g, 0)),
            full((1, 128)),
            full((1, 128)),
            full((128, ITEM_SIZE)),
            full((1, ITEM_SIZE)),
            full((16, D_SRC)),
            pl.BlockSpec((IBLK, 128), lambda g: (g, 0)),
            full((ITEM_SIZE, ITEM_SIZE)),
            full((1, ITEM_SIZE)),
            pl.BlockSpec((1, 1, IBLK), lambda g: (g, 0, 0)),
        ],
        out_specs=[
            pl.BlockSpec((BB, L, ITEM_SIZE), lambda g: (g, 0, 0)),
            pl.BlockSpec((BB, L, D_SRC), lambda g: (g, 0, 0)),
            pl.BlockSpec((BB, L, NC, ITEM_SIZE), lambda g: (g, 0, 0, 0)),
            pl.BlockSpec((1, 1, IBLK), lambda g: (g, 0, 0)),
        ],
        out_shape=[
            jax.ShapeDtypeStruct((B, L, ITEM_SIZE), f32),
            jax.ShapeDtypeStruct((B, L, D_SRC), f32),
            jax.ShapeDtypeStruct((B, L, NC, ITEM_SIZE), f32),
            jax.ShapeDtypeStruct((GRID, 1, IBLK), jnp.bool_),
        ],
    )(qcat, ids16, emb0p, cmask, wq, bq, qsrct, item, wi, bi, iidm3d)


def kernel(query_id, search_source, click_item_id, click_item_category,
           query_words, item_id_emb, item_cat_emb, query_id_emb, qsrc_emb,
           qword_emb, W_q, b_q, W_i, b_i):
    i32 = jnp.int32
    f32 = jnp.float32
    qid2d = query_id.astype(i32).reshape(T)
    qw2d = query_words.astype(i32).reshape(T * NW)
    ii2d = click_item_id.astype(i32).reshape(T * NC)
    ic2d = click_item_category.astype(i32).reshape(T * NC)

    qcat, item_rows = _sc_gather(
        qid2d, qw2d, ii2d, ic2d,
        query_id_emb, qword_emb, item_id_emb, item_cat_emb)

    ids16 = jnp.concatenate(
        [query_words.astype(i32).reshape(T, NW),
         search_source.astype(i32).reshape(T, 1),
         jnp.zeros((T, 16 - NW - 1), i32)], axis=1)
    iidm3d = click_item_id.astype(i32).reshape(GRID, 1, IBLK)
    emb0p = jnp.concatenate(
        [jnp.zeros((1, D_ID), f32), qword_emb[0:1, :]], axis=1)
    cmask = jnp.concatenate(
        [jnp.zeros((1, D_ID), f32), jnp.ones((1, D_W), f32)], axis=1)
    qsrct = jnp.concatenate(
        [qsrc_emb, jnp.zeros((16 - qsrc_emb.shape[0], D_SRC), f32)], axis=0)
    query_emb, q_src_e, item_out, mask3d = _tc_project(
        qcat, ids16, emb0p, cmask, W_q, b_q.reshape(1, ITEM_SIZE), qsrct,
        item_rows, W_i, b_i.reshape(1, ITEM_SIZE), iidm3d)

    return (query_emb,
            q_src_e,
            item_out,
            mask3d.reshape(B, L, NC))
